# baseline (speedup 1.0000x reference)
import jax
import jax.numpy as jnp
from jax import lax
from jax.experimental import pallas as pl
from jax.experimental.pallas import tpu as pltpu
from jax.experimental.pallas import tpu_sc as plsc

H = 512
HC = H // 16
N_NODES = 50000
NC = 2
NS = 16
NW = NC * NS
N_OUT_PAD = 50176
NPW = N_OUT_PAD // NW
W = 112
BATCH = 128
SEG_BIG = 0x3FFFFFFF
BS_ITERS = 14


def _sc_body(ids_hbm, seg_hbm, table_hbm, zeros_hbm, out_hbm,
             probe_v, ids_v, seg_v, rows_v, stage_v, acc_v, sem):
    c = lax.axis_index("c")
    s = lax.axis_index("s")
    wid = c * NS + s
    wlo = wid * NPW
    wend = wlo + NPW
    nchunk = seg_hbm.shape[0] // 16

    def lower_bound(bval):
        def step(_, lohi):
            lo, hi = lohi
            m = (lo + hi) // 2
            pltpu.sync_copy(seg_hbm.at[pl.ds(m * 16, 16)], probe_v)
            pred = probe_v[...][0] < bval
            return (jnp.where(pred, m + 1, lo), jnp.where(pred, hi, m))

        lo, _ = lax.fori_loop(0, BS_ITERS, step,
                              (jnp.int32(0), jnp.int32(nchunk)))
        cm1 = jnp.maximum(lo - 1, 0)
        pltpu.sync_copy(seg_hbm.at[pl.ds(cm1 * 16, 16)], probe_v)
        x = probe_v[...]
        cnt = jnp.int32(0)
        for j in range(16):
            cnt = cnt + jnp.where(x[j] < bval, 1, 0).astype(jnp.int32)
        return jnp.where(lo == 0, 0, (lo - 1) * 16 + cnt)

    st0 = lower_bound(wlo)
    end_w = lower_bound(wend)
    base = (st0 // 8) * 8
    nb = (end_w - base + (BATCH - 1)) // BATCH

    pltpu.sync_copy(zeros_hbm, stage_v.at[pl.ds(0, W)])
    zvec = jnp.zeros((16,), jnp.float32)
    for k in range(HC):
        acc_v[pl.ds(k * 16, 16)] = zvec

    def flush_n(win_base, n_fl):
        @pl.when(n_fl > 0)
        def _():
            def fbody(f, _):
                wb = pl.multiple_of(win_base + f * W, 8)
                pltpu.sync_copy(stage_v.at[pl.ds(0, W)],
                                out_hbm.at[pl.ds(wb, W)])
                pltpu.sync_copy(zeros_hbm, stage_v.at[pl.ds(0, W)])
                return 0

            lax.fori_loop(0, n_fl, fbody, 0)

    def batch_body(ptr, win_base):
        st = base + ptr * BATCH
        pltpu.sync_copy(ids_hbm.at[pl.ds(st, BATCH)], ids_v)
        pltpu.sync_copy(seg_hbm.at[pl.ds(st, BATCH + 16)], seg_v)
        pltpu.async_copy(table_hbm.at[ids_v], rows_v, sem).wait()

        def chunk(m, win_base):
            sv = seg_v[pl.ds(m * 16, 16)]
            sw = seg_v[pl.ds(m * 16 + 16, 16)]
            accs = [acc_v[pl.ds(k * 16, 16)] for k in range(HC)]
            for j in range(16):
                seg_r = sv[j]
                seg_n = sw[0] if j == 15 else sv[j + 1]
                n_fl = jnp.clip((seg_r - win_base) // W, 0,
                                (wend - win_base) // W)
                flush_n(win_base, n_fl)
                win_base = win_base + n_fl * W
                rel = seg_r - win_base
                is_end = seg_r != seg_n
                valid = is_end & (rel >= 0) & (rel < W)
                relc = jnp.where(valid, rel, W)
                keepv = jnp.broadcast_to(
                    jnp.where(is_end, jnp.float32(0), jnp.float32(1)), (16,))
                for k in range(HC):
                    row_k = rows_v[m * 16 + j, pl.ds(k * 16, 16)]
                    sum_k = accs[k] + row_k
                    stage_v[relc, pl.ds(k * 16, 16)] = sum_k
                    accs[k] = sum_k * keepv
            for k in range(HC):
                acc_v[pl.ds(k * 16, 16)] = accs[k]
            return win_base

        return lax.fori_loop(0, BATCH // 16, chunk, win_base)

    win_base = lax.fori_loop(0, nb, batch_body, wlo)

    flush_n(win_base, (wend - win_base) // W)


@jax.jit
def _impl(ids32, seg32, table):
    n_sub = ids32.shape[0]
    pad = 2 * BATCH + ((-(n_sub + 2 * BATCH)) % BATCH)
    ids_p = jnp.concatenate([ids32, jnp.zeros((pad,), jnp.int32)])
    seg_p = jnp.concatenate([seg32, jnp.full((pad,), SEG_BIG, jnp.int32)])
    zeros = jnp.zeros((W, H), jnp.float32)

    mesh = plsc.VectorSubcoreMesh(core_axis_name="c", subcore_axis_name="s")
    run = pl.kernel(
        _sc_body,
        out_type=jax.ShapeDtypeStruct((N_OUT_PAD, H), jnp.float32),
        mesh=mesh,
        scratch_types=[
            pltpu.VMEM((16,), jnp.int32),
            pltpu.VMEM((BATCH,), jnp.int32),
            pltpu.VMEM((BATCH + 16,), jnp.int32),
            pltpu.VMEM((BATCH, H), jnp.float32),
            pltpu.VMEM((W + 1, H), jnp.float32),
            pltpu.VMEM((H,), jnp.float32),
            pltpu.SemaphoreType.DMA,
        ],
    )
    return run(ids_p, seg_p, table, zeros)


def kernel(subtoken_ids, segment_ids, n_nodes, table):
    del n_nodes
    ids32 = subtoken_ids.astype(jnp.int32)
    seg32 = segment_ids.astype(jnp.int32)
    out = _impl(ids32, seg32, table)
    return out[:N_NODES]


# speedup vs baseline: 1.6502x; 1.6502x over previous
import jax
import jax.numpy as jnp
from jax import lax
from jax.experimental import pallas as pl
from jax.experimental.pallas import tpu as pltpu
from jax.experimental.pallas import tpu_sc as plsc

H = 512
HC = H // 16
N_NODES = 50000
NC = 2
NS = 16
NW = NC * NS
N_OUT_PAD = 53248
NPW = N_OUT_PAD // NW
W = 128
BATCH = 112
SEG_BIG = 0x3FFFFFFF
BS_ITERS = 14


def _sc_body(ids_hbm, seg_hbm, table_hbm, zeros_hbm, out_hbm,
             probe_v, ids_v, seg_v, rows_v, stage_v, acc_v, sem):
    c = lax.axis_index("c")
    s = lax.axis_index("s")
    wid = c * NS + s
    wlo = wid * NPW
    wend = wlo + NPW
    nchunk = seg_hbm.shape[0] // 16

    def lower_bound(bval):
        def step(_, lohi):
            lo, hi = lohi
            m = (lo + hi) // 2
            pltpu.sync_copy(seg_hbm.at[pl.ds(m * 16, 16)], probe_v)
            pred = probe_v[...][0] < bval
            return (jnp.where(pred, m + 1, lo), jnp.where(pred, hi, m))

        lo, _ = lax.fori_loop(0, BS_ITERS, step,
                              (jnp.int32(0), jnp.int32(nchunk)))
        cm1 = jnp.maximum(lo - 1, 0)
        pltpu.sync_copy(seg_hbm.at[pl.ds(cm1 * 16, 16)], probe_v)
        x = probe_v[...]
        cnt = jnp.int32(0)
        for j in range(16):
            cnt = cnt + jnp.where(x[j] < bval, 1, 0).astype(jnp.int32)
        return jnp.where(lo == 0, 0, (lo - 1) * 16 + cnt)

    st0 = lower_bound(wlo)
    end_w = lower_bound(wend)
    base = (st0 // 8) * 8
    nb = (end_w - base + (BATCH - 1)) // BATCH

    pltpu.sync_copy(zeros_hbm, stage_v.at[pl.ds(0, W)])
    zvec = jnp.zeros((16,), jnp.float32)
    for k in range(HC):
        acc_v[pl.ds(k * 16, 16)] = zvec

    def flush_n(win_base, n_fl):
        @pl.when(n_fl > 0)
        def _():
            def fbody(f, _):
                wb = pl.multiple_of(win_base + f * W, 8)
                pltpu.sync_copy(stage_v.at[pl.ds(0, W)],
                                out_hbm.at[pl.ds(wb, W)])
                pltpu.sync_copy(zeros_hbm, stage_v.at[pl.ds(0, W)])
                return 0

            lax.fori_loop(0, n_fl, fbody, 0)

    def batch_body(ptr, win_base):
        st = base + ptr * BATCH
        pltpu.sync_copy(ids_hbm.at[pl.ds(st, BATCH)], ids_v)
        pltpu.sync_copy(seg_hbm.at[pl.ds(st, BATCH + 16)], seg_v)
        pltpu.async_copy(table_hbm.at[ids_v], rows_v, sem).wait()

        def chunk(m, win_base):
            sv = seg_v[pl.ds(m * 16, 16)]
            sw = seg_v[pl.ds(m * 16 + 16, 16)]
            accs = [acc_v[pl.ds(k * 16, 16)] for k in range(HC)]
            for j in range(16):
                seg_r = sv[j]
                seg_n = sw[0] if j == 15 else sv[j + 1]
                n_fl = jnp.clip(jnp.maximum(seg_r - win_base, 0) // W, 0,
                                (wend - win_base) // W)
                flush_n(win_base, n_fl)
                win_base = win_base + n_fl * W
                rel = seg_r - win_base
                is_end = seg_r != seg_n
                valid = is_end & (rel >= 0) & (rel < W)
                keepv = jnp.broadcast_to(
                    jnp.where(is_end, jnp.float32(0), jnp.float32(1)), (16,))
                sums = []
                for k in range(HC):
                    row_k = rows_v[m * 16 + j, pl.ds(k * 16, 16)]
                    sums.append(accs[k] + row_k)

                @pl.when(valid)
                def _(rel=rel, sums=sums):
                    for k in range(HC):
                        stage_v[rel, pl.ds(k * 16, 16)] = sums[k]

                for k in range(HC):
                    accs[k] = sums[k] * keepv
            for k in range(HC):
                acc_v[pl.ds(k * 16, 16)] = accs[k]
            return win_base

        return lax.fori_loop(0, BATCH // 16, chunk, win_base)

    win_base = lax.fori_loop(0, nb, batch_body, wlo)

    flush_n(win_base, (wend - win_base) // W)


@jax.jit
def _impl(ids32, seg32, table):
    n_sub = ids32.shape[0]
    pad = 2 * BATCH + ((-(n_sub + 2 * BATCH)) % BATCH)
    ids_p = jnp.concatenate([ids32, jnp.zeros((pad,), jnp.int32)])
    seg_p = jnp.concatenate([seg32, jnp.full((pad,), SEG_BIG, jnp.int32)])
    zeros = jnp.zeros((W, H), jnp.float32)

    mesh = plsc.VectorSubcoreMesh(core_axis_name="c", subcore_axis_name="s")
    run = pl.kernel(
        _sc_body,
        out_type=jax.ShapeDtypeStruct((N_OUT_PAD, H), jnp.float32),
        mesh=mesh,
        scratch_types=[
            pltpu.VMEM((16,), jnp.int32),
            pltpu.VMEM((BATCH,), jnp.int32),
            pltpu.VMEM((BATCH + 16,), jnp.int32),
            pltpu.VMEM((BATCH, H), jnp.float32),
            pltpu.VMEM((W, H), jnp.float32),
            pltpu.VMEM((H,), jnp.float32),
            pltpu.SemaphoreType.DMA,
        ],
    )
    return run(ids_p, seg_p, table, zeros)


def kernel(subtoken_ids, segment_ids, n_nodes, table):
    del n_nodes
    ids32 = subtoken_ids.astype(jnp.int32)
    seg32 = segment_ids.astype(jnp.int32)
    out = _impl(ids32, seg32, table)
    return out[:N_NODES]


# speedup vs baseline: 3.3023x; 2.0012x over previous
import jax
import jax.numpy as jnp
from jax import lax
from jax.experimental import pallas as pl
from jax.experimental.pallas import tpu as pltpu
from jax.experimental.pallas import tpu_sc as plsc

H = 512
HC = H // 16
N_NODES = 50000
NC = 2
NS = 16
NW = NC * NS
N_OUT_PAD = 53248
NPW = N_OUT_PAD // NW
W = 128
BATCH = 112
SEG_BIG = 0x3FFFFFFF
BS_ITERS = 14


def _sc_body(ids_hbm, seg_hbm, table_hbm, zeros_hbm, out_hbm,
             probe_v, ids_v, seg_v, rows_v, stage_v, acc_v, sem):
    c = lax.axis_index("c")
    s = lax.axis_index("s")
    wid = c * NS + s
    wlo = wid * NPW
    wend = wlo + NPW
    nchunk = seg_hbm.shape[0] // 16

    def lower_bound(bval):
        def step(_, lohi):
            lo, hi = lohi
            m = (lo + hi) // 2
            pltpu.sync_copy(seg_hbm.at[pl.ds(m * 16, 16)], probe_v)
            pred = probe_v[...][0] < bval
            return (jnp.where(pred, m + 1, lo), jnp.where(pred, hi, m))

        lo, _ = lax.fori_loop(0, BS_ITERS, step,
                              (jnp.int32(0), jnp.int32(nchunk)))
        cm1 = jnp.maximum(lo - 1, 0)
        pltpu.sync_copy(seg_hbm.at[pl.ds(cm1 * 16, 16)], probe_v)
        x = probe_v[...]
        cnt = jnp.int32(0)
        for j in range(16):
            cnt = cnt + jnp.where(x[j] < bval, 1, 0).astype(jnp.int32)
        return jnp.where(lo == 0, 0, (lo - 1) * 16 + cnt)

    st0 = lower_bound(wlo)
    end_w = lower_bound(wend)
    base = (st0 // 8) * 8
    nb = (end_w - base + (BATCH - 1)) // BATCH

    pltpu.sync_copy(zeros_hbm, stage_v.at[pl.ds(0, W)])
    zvec = jnp.zeros((16,), jnp.float32)
    for k in range(HC):
        acc_v[pl.ds(k * 16, 16)] = zvec

    def flush_n(win_base, n_fl):
        @pl.when(n_fl > 0)
        def _():
            def fbody(f, _):
                wb = pl.multiple_of(win_base + f * W, 8)
                pltpu.sync_copy(stage_v.at[pl.ds(0, W)],
                                out_hbm.at[pl.ds(wb, W)])
                pltpu.sync_copy(zeros_hbm, stage_v.at[pl.ds(0, W)])
                return 0

            lax.fori_loop(0, n_fl, fbody, 0)

    def batch_body(ptr, win_base):
        st = base + ptr * BATCH
        pltpu.sync_copy(ids_hbm.at[pl.ds(st, BATCH)], ids_v)
        pltpu.sync_copy(seg_hbm.at[pl.ds(st, BATCH + 16)], seg_v)
        pltpu.async_copy(table_hbm.at[ids_v], rows_v, sem).wait()

        def process(m, win_base, do_flush):
            sv = seg_v[pl.ds(m * 16, 16)]
            sw = seg_v[pl.ds(m * 16 + 16, 16)]
            accs = [acc_v[pl.ds(k * 16, 16)] for k in range(HC)]
            for j in range(16):
                seg_r = sv[j]
                seg_n = sw[0] if j == 15 else sv[j + 1]
                if do_flush:
                    n_fl = jnp.clip(jnp.maximum(seg_r - win_base, 0) // W, 0,
                                    (wend - win_base) // W)
                    flush_n(win_base, n_fl)
                    win_base = win_base + n_fl * W
                rel = seg_r - win_base
                is_end = seg_r != seg_n
                valid = is_end & (rel >= 0) & (rel < W)
                keepv = jnp.broadcast_to(
                    jnp.where(is_end, jnp.float32(0), jnp.float32(1)), (16,))
                sums = []
                for k in range(HC):
                    row_k = rows_v[m * 16 + j, pl.ds(k * 16, 16)]
                    sums.append(accs[k] + row_k)

                @pl.when(valid)
                def _(rel=rel, sums=sums):
                    for k in range(HC):
                        stage_v[rel, pl.ds(k * 16, 16)] = sums[k]

                for k in range(HC):
                    accs[k] = sums[k] * keepv
            for k in range(HC):
                acc_v[pl.ds(k * 16, 16)] = accs[k]
            return win_base

        def chunk(m, win_base):
            last1 = seg_v[pl.ds(m * 16, 16)][15]
            return lax.cond(
                last1 < win_base + W,
                lambda wb: process(m, wb, False),
                lambda wb: process(m, wb, True),
                win_base)

        return lax.fori_loop(0, BATCH // 16, chunk, win_base)

    win_base = lax.fori_loop(0, nb, batch_body, wlo)

    flush_n(win_base, (wend - win_base) // W)


@jax.jit
def _impl(ids32, seg32, table):
    n_sub = ids32.shape[0]
    pad = 2 * BATCH + ((-(n_sub + 2 * BATCH)) % BATCH)
    ids_p = jnp.concatenate([ids32, jnp.zeros((pad,), jnp.int32)])
    seg_p = jnp.concatenate([seg32, jnp.full((pad,), SEG_BIG, jnp.int32)])
    zeros = jnp.zeros((W, H), jnp.float32)

    mesh = plsc.VectorSubcoreMesh(core_axis_name="c", subcore_axis_name="s")
    run = pl.kernel(
        _sc_body,
        out_type=jax.ShapeDtypeStruct((N_OUT_PAD, H), jnp.float32),
        mesh=mesh,
        scratch_types=[
            pltpu.VMEM((16,), jnp.int32),
            pltpu.VMEM((BATCH,), jnp.int32),
            pltpu.VMEM((BATCH + 16,), jnp.int32),
            pltpu.VMEM((BATCH, H), jnp.float32),
            pltpu.VMEM((W, H), jnp.float32),
            pltpu.VMEM((H,), jnp.float32),
            pltpu.SemaphoreType.DMA,
        ],
    )
    return run(ids_p, seg_p, table, zeros)


def kernel(subtoken_ids, segment_ids, n_nodes, table):
    del n_nodes
    ids32 = subtoken_ids.astype(jnp.int32)
    seg32 = segment_ids.astype(jnp.int32)
    out = _impl(ids32, seg32, table)
    return out[:N_NODES]
